# Initial kernel scaffold; baseline (speedup 1.0000x reference)
import jax
import jax.numpy as jnp
from jax.experimental import pallas as pl


def kernel(x, edge_index, Wn1, Ws1, b1, Wn2, Ws2, b2, Wn3, Ws3, b3):
    raise NotImplementedError("write your pallas kernel here")


# speedup vs baseline: 4.0319x; 4.0319x over previous
import functools

import jax
import jax.numpy as jnp
from jax import lax
from jax.experimental import pallas as pl
from jax.experimental.pallas import tpu as pltpu
from jax.experimental.pallas import tpu_sc as plsc

N = 10000
E = 320000
D = 128

NC = 2
NS = 16
NW = NC * NS
EPW = E // NW
K = 80
NBLK = EPW // K
CH = 8
NCHK = N // CH
CPT = (NCHK + NS - 1) // NS


def _zero_acc(s, acc_sh, zbuf):
    zvec = jnp.zeros((16,), jnp.float32)

    def fill_z(i, _):
        for j in range(8):
            zbuf[i, pl.ds(j * 16, 16)] = zvec
        return 0

    lax.fori_loop(0, CH, fill_z, 0)

    def zero_chunk(t, _):
        chunk = jnp.minimum(s + t * NS, NCHK - 1)
        pltpu.sync_copy(zbuf, acc_sh.at[pl.ds(chunk * CH, CH)])
        return 0

    lax.fori_loop(0, CPT, zero_chunk, 0)


def _drain_acc(c, s, acc_sh, out_hbm):
    def drain_chunk(t, _):
        chunk = jnp.minimum(s + t * NS, NCHK - 1)
        r0 = chunk * CH
        pltpu.sync_copy(acc_sh.at[pl.ds(r0, CH)], out_hbm.at[c, pl.ds(r0, CH)])
        return 0

    lax.fori_loop(0, CPT, drain_chunk, 0)


def _sc_agg_body(h_hbm, src_hbm, dst_hbm, out_hbm, acc_sh, sidx, didx, rows,
                 zbuf, sem):
    c = lax.axis_index("c")
    s = lax.axis_index("s")
    wid = s * NC + c

    _zero_acc(s, acc_sh, zbuf)
    plsc.subcore_barrier()

    ebase = wid * EPW

    def edge_blk(b, _):
        off = ebase + b * K
        pltpu.sync_copy(src_hbm.at[pl.ds(off, K)], sidx)
        pltpu.sync_copy(dst_hbm.at[pl.ds(off, K)], didx)
        pltpu.async_copy(h_hbm.at[sidx], rows, sem).wait()
        pltpu.sync_copy(rows, acc_sh.at[didx], add=True)
        return 0

    lax.fori_loop(0, NBLK, edge_blk, 0)
    plsc.subcore_barrier()

    _drain_acc(c, s, acc_sh, out_hbm)


def _sc_deg_body(dst_hbm, out_hbm, acc_sh, didx, ones, zbuf, sem):
    c = lax.axis_index("c")
    s = lax.axis_index("s")
    wid = s * NC + c

    _zero_acc(s, acc_sh, zbuf)

    ovec = jnp.ones((16,), jnp.float32)

    def fill_o(i, _):
        for j in range(8):
            ones[i, pl.ds(j * 16, 16)] = ovec
        return 0

    lax.fori_loop(0, K, fill_o, 0)
    plsc.subcore_barrier()

    ebase = wid * EPW

    def edge_blk(b, _):
        off = ebase + b * K
        pltpu.sync_copy(dst_hbm.at[pl.ds(off, K)], didx)
        pltpu.sync_copy(ones, acc_sh.at[didx], add=True)
        return 0

    lax.fori_loop(0, NBLK, edge_blk, 0)
    plsc.subcore_barrier()

    _drain_acc(c, s, acc_sh, out_hbm)


_MESH = plsc.VectorSubcoreMesh(core_axis_name="c", subcore_axis_name="s",
                               num_cores=NC, num_subcores=NS)


def _sc_aggregate(h, src, dst):
    kern = pl.kernel(
        _sc_agg_body,
        out_type=jax.ShapeDtypeStruct((NC, N, D), jnp.float32),
        mesh=_MESH,
        scratch_types=[
            pltpu.VMEM_SHARED((N, D), jnp.float32),
            pltpu.VMEM((K,), jnp.int32),
            pltpu.VMEM((K,), jnp.int32),
            pltpu.VMEM((K, D), jnp.float32),
            pltpu.VMEM((CH, D), jnp.float32),
            pltpu.SemaphoreType.DMA,
        ],
        name="sage_sc_agg",
    )
    return kern(h, src, dst)


def _sc_degree(dst):
    kern = pl.kernel(
        _sc_deg_body,
        out_type=jax.ShapeDtypeStruct((NC, N, D), jnp.float32),
        mesh=_MESH,
        scratch_types=[
            pltpu.VMEM_SHARED((N, D), jnp.float32),
            pltpu.VMEM((K,), jnp.int32),
            pltpu.VMEM((K, D), jnp.float32),
            pltpu.VMEM((CH, D), jnp.float32),
            pltpu.SemaphoreType.DMA,
        ],
        name="sage_sc_deg",
    )
    return kern(dst)


def _tc_layer_body(relu, h_ref, a0_ref, a1_ref, d0_ref, d1_ref, ws_ref,
                   wn_ref, b_ref, o_ref):
    deg = jnp.maximum(d0_ref[:, 0:1] + d1_ref[:, 0:1], 1.0)
    mean = (a0_ref[...] + a1_ref[...]) / deg
    out = (jnp.dot(h_ref[...], ws_ref[...], preferred_element_type=jnp.float32)
           + jnp.dot(mean, wn_ref[...], preferred_element_type=jnp.float32)
           + b_ref[...])
    if relu:
        out = jnp.maximum(out, 0.0)
    o_ref[...] = out


def _tc_layer(h, A, degp, Ws, Wn, b, relu):
    F = Ws.shape[1]
    BN = 1000
    grid = (N // BN,)
    out = pl.pallas_call(
        functools.partial(_tc_layer_body, relu),
        grid=grid,
        in_specs=[
            pl.BlockSpec((BN, D), lambda i: (i, 0)),
            pl.BlockSpec((BN, D), lambda i: (i, 0)),
            pl.BlockSpec((BN, D), lambda i: (i, 0)),
            pl.BlockSpec((BN, D), lambda i: (i, 0)),
            pl.BlockSpec((BN, D), lambda i: (i, 0)),
            pl.BlockSpec((D, F), lambda i: (0, 0)),
            pl.BlockSpec((D, F), lambda i: (0, 0)),
            pl.BlockSpec((1, F), lambda i: (0, 0)),
        ],
        out_specs=pl.BlockSpec((BN, F), lambda i: (i, 0)),
        out_shape=jax.ShapeDtypeStruct((N, F), jnp.float32),
        name="sage_tc_layer",
    )(h, A[0], A[1], degp[0], degp[1], Ws, Wn, b.reshape(1, F))
    return out


def kernel(x, edge_index, Wn1, Ws1, b1, Wn2, Ws2, b2, Wn3, Ws3, b3):
    src = edge_index[0]
    dst = edge_index[1]
    degp = _sc_degree(dst)
    A1 = _sc_aggregate(x, src, dst)
    h1 = _tc_layer(x, A1, degp, Ws1, Wn1, b1, relu=True)
    A2 = _sc_aggregate(h1, src, dst)
    h2 = _tc_layer(h1, A2, degp, Ws2, Wn2, b2, relu=True)
    A3 = _sc_aggregate(h2, src, dst)
    out = _tc_layer(h2, A3, degp, Ws3, Wn3, b3, relu=False)
    return out
